# baseline (speedup 1.0000x reference)
import functools

import numpy as np
import jax
import jax.numpy as jnp
from jax import lax
from jax.experimental import pallas as pl
from jax.experimental.pallas import tpu as pltpu
from jax.experimental.pallas import tpu_sc as plsc

EMBED = 64
SEQ = 200
LANES = 16
NUM_WORKERS = 32
CHUNKS = ((0, 104), (104, 96))


def _positional_encoding_np(length, depth):
    half = depth / 2
    positions = np.arange(length)[:, np.newaxis]
    depths = np.arange(half)[np.newaxis, :] / half
    angle_rates = 1 / 10000 ** depths
    angle_rads = positions * angle_rates
    return np.concatenate(
        [np.sin(angle_rads), np.cos(angle_rads)], axis=-1
    ).astype(np.float32)


def _make_sc_kernel(batch):
    assert batch % NUM_WORKERS == 0
    b_per_w = batch // NUM_WORKERS
    scale = float(EMBED) ** 0.5

    @functools.partial(
        pl.kernel,
        mesh=plsc.VectorSubcoreMesh(core_axis_name="c", subcore_axis_name="s"),
        out_type=jax.ShapeDtypeStruct((batch * SEQ, EMBED), jnp.float32),
        scratch_types=[
            pltpu.VMEM((SEQ,), jnp.int32),
            pltpu.VMEM((SEQ, EMBED), jnp.float32),
            pltpu.VMEM((SEQ, EMBED), jnp.float32),
            pltpu.SemaphoreType.DMA,
        ],
        compiler_params=pltpu.CompilerParams(use_tc_tiling_on_sc=False),
    )
    def sc_kernel(x_hbm, table_hbm, pe_hbm, out_hbm, idx_v, rows_v, pe_v, sem):
        wid = lax.axis_index("s") * 2 + lax.axis_index("c")
        pltpu.sync_copy(pe_hbm, pe_v)

        def per_batch(i, carry):
            b = wid * b_per_w + i
            pltpu.sync_copy(x_hbm.at[pl.ds(b * SEQ, SEQ)], idx_v)
            copies = [
                pltpu.async_copy(
                    table_hbm.at[idx_v.at[pl.ds(off, n)]],
                    rows_v.at[pl.ds(off, n)],
                    sem,
                )
                for off, n in CHUNKS
            ]
            for cp in copies:
                cp.wait()

            def per_row(r, c2):
                for c in range(EMBED // LANES):
                    sl = pl.ds(c * LANES, LANES)
                    rows_v[r, sl] = rows_v[r, sl] * scale + pe_v[r, sl]
                return c2

            lax.fori_loop(0, SEQ, per_row, 0)
            pltpu.sync_copy(rows_v, out_hbm.at[pl.ds(b * SEQ, SEQ)])
            return carry

        lax.fori_loop(0, b_per_w, per_batch, 0)

    return sc_kernel


def kernel(x, table):
    batch, seq = x.shape
    assert seq == SEQ and table.shape[1] == EMBED
    pe = jnp.asarray(_positional_encoding_np(SEQ, EMBED))
    x_flat = x.reshape(-1).astype(jnp.int32)
    out = _make_sc_kernel(batch)(x_flat, table, pe)
    return out.reshape(batch, SEQ, EMBED)
